# baseline (speedup 1.0000x reference)
import dataclasses
import functools

import numpy as np
import jax
import jax.numpy as jnp
from jax import lax
from jax.experimental import pallas as pl
from jax.experimental.pallas import tpu as pltpu
from jax.experimental.pallas import tpu_sc as plsc

_SHUFFLE_CHANNEL = 2048
_TOTAL = 8192
_NC, _NS, _L = 2, 16, 16
_NW = _NC * _NS
_ROWS = 4 * 1024
_RPW = _ROWS // _NW


def _build_index() -> np.ndarray:
    pkey = jax.random.key(42)
    random_sort = jax.random.permutation(pkey, _TOTAL)[:_SHUFFLE_CHANNEL]
    random_index = jnp.sort(random_sort)
    idx = jnp.arange(_TOTAL, dtype=random_sort.dtype).at[random_index].set(random_sort)
    return np.asarray(idx).astype(np.int32)


_IDX = _build_index()


_RBLK = 2


@jax.jit
def _sc_shuffle(x2d, idx):
    mesh = plsc.VectorSubcoreMesh(
        core_axis_name="c", subcore_axis_name="s",
        num_cores=_NC, num_subcores=_NS,
    )

    cp = pltpu.CompilerParams()
    if "needs_layout_passes" in pltpu.CompilerParams.__dataclass_fields__:
        cp = dataclasses.replace(cp, needs_layout_passes=False)

    @functools.partial(
        pl.kernel,
        mesh=mesh,
        compiler_params=cp,
        out_type=jax.ShapeDtypeStruct((_ROWS, _TOTAL), jnp.float32),
        scratch_types=[
            pltpu.VMEM((_TOTAL,), jnp.int32),
        ],
    )
    def k(x_hbm, idx_hbm, o_hbm, idx_v):
        pltpu.sync_copy(idx_hbm, idx_v)

        def body(in_v, out_v):
            for r in range(_RBLK):
                rsplat = jnp.full((_L,), r, jnp.int32)

                @pl.loop(0, _TOTAL, step=_L, unroll=4)
                def _chunk(j):
                    iv = idx_v[pl.ds(j, _L)]
                    out_v[r, pl.ds(j, _L)] = plsc.load_gather(in_v, [rsplat, iv])

        pltpu.emit_pipeline(
            body,
            grid=(_ROWS // _RBLK,),
            in_specs=[pl.BlockSpec((_RBLK, _TOTAL), lambda i: (i, 0))],
            out_specs=[pl.BlockSpec((_RBLK, _TOTAL), lambda i: (i, 0))],
            core_axis_name=("c", "s"),
            dimension_semantics=(pltpu.PARALLEL,),
        )(x_hbm, o_hbm)

    return k(x2d, idx)


def kernel(input):
    x2d = input.reshape(_ROWS, _TOTAL)
    out = _sc_shuffle(x2d, jnp.asarray(_IDX))
    return out.reshape(input.shape)


# speedup vs baseline: 2.5724x; 2.5724x over previous
import dataclasses
import functools

import numpy as np
import jax
import jax.numpy as jnp
from jax import lax
from jax.experimental import pallas as pl
from jax.experimental.pallas import tpu as pltpu
from jax.experimental.pallas import tpu_sc as plsc

_SHUFFLE_CHANNEL = 2048
_TOTAL = 8192
_NC, _NS, _L = 2, 16, 16
_NW = _NC * _NS
_ROWS = 4 * 1024
_RPW = _ROWS // _NW


def _build_index() -> np.ndarray:
    pkey = jax.random.key(42)
    random_sort = jax.random.permutation(pkey, _TOTAL)[:_SHUFFLE_CHANNEL]
    random_index = jnp.sort(random_sort)
    idx = jnp.arange(_TOTAL, dtype=random_sort.dtype).at[random_index].set(random_sort)
    return np.asarray(idx).astype(np.int32)


_IDX = _build_index()


_RBLK = 2


@jax.jit
def _sc_shuffle(x2d, idx):
    mesh = plsc.VectorSubcoreMesh(
        core_axis_name="c", subcore_axis_name="s",
        num_cores=_NC, num_subcores=_NS,
    )

    cp = pltpu.CompilerParams()
    if "needs_layout_passes" in pltpu.CompilerParams.__dataclass_fields__:
        cp = dataclasses.replace(cp, needs_layout_passes=False)

    @functools.partial(
        pl.kernel,
        mesh=mesh,
        compiler_params=cp,
        out_type=jax.ShapeDtypeStruct((_ROWS, _TOTAL), jnp.float32),
        scratch_types=[
            pltpu.VMEM((_TOTAL,), jnp.int32),
            pltpu.VMEM((_TOTAL,), jnp.float32),
            pltpu.VMEM((_TOTAL,), jnp.float32),
        ],
    )
    def k(x_hbm, idx_hbm, o_hbm, idx_v, in_v, out_v):
        wid = lax.axis_index("s") * _NC + lax.axis_index("c")
        base = wid * _RPW
        pltpu.sync_copy(idx_hbm, idx_v)

        @pl.loop(0, _RPW)
        def _row(i):
            row = base + i
            pltpu.sync_copy(x_hbm.at[row], in_v)

            @plsc.parallel_loop(0, _TOTAL, step=_L, unroll=8)
            def _chunk(j):
                iv = idx_v[pl.ds(j, _L)]
                out_v[pl.ds(j, _L)] = plsc.load_gather(in_v, [iv])

            pltpu.sync_copy(out_v, o_hbm.at[row])

    return k(x2d, idx)


def kernel(input):
    x2d = input.reshape(_ROWS, _TOTAL)
    out = _sc_shuffle(x2d, jnp.asarray(_IDX))
    return out.reshape(input.shape)


# speedup vs baseline: 2.9953x; 1.1644x over previous
import dataclasses
import functools

import numpy as np
import jax
import jax.numpy as jnp
from jax import lax
from jax.experimental import pallas as pl
from jax.experimental.pallas import tpu as pltpu
from jax.experimental.pallas import tpu_sc as plsc

_SHUFFLE_CHANNEL = 2048
_TOTAL = 8192
_NC, _NS, _L = 2, 16, 16
_NW = _NC * _NS
_ROWS = 4 * 1024
_RPW = _ROWS // _NW


def _build_index() -> np.ndarray:
    pkey = jax.random.key(42)
    random_sort = jax.random.permutation(pkey, _TOTAL)[:_SHUFFLE_CHANNEL]
    random_index = jnp.sort(random_sort)
    rs = np.asarray(random_sort).astype(np.int32)
    ri = np.asarray(random_index).astype(np.int32)
    return np.concatenate([rs, ri])


_IDX = _build_index()


_RBLK = 2


@jax.jit
def _sc_shuffle(x2d, idx):
    mesh = plsc.VectorSubcoreMesh(
        core_axis_name="c", subcore_axis_name="s",
        num_cores=_NC, num_subcores=_NS,
    )

    cp = pltpu.CompilerParams()
    if "needs_layout_passes" in pltpu.CompilerParams.__dataclass_fields__:
        cp = dataclasses.replace(cp, needs_layout_passes=False)

    @functools.partial(
        pl.kernel,
        mesh=mesh,
        compiler_params=cp,
        out_type=jax.ShapeDtypeStruct((_ROWS, _TOTAL), jnp.float32),
        scratch_types=[
            pltpu.VMEM((2 * _SHUFFLE_CHANNEL,), jnp.int32),
            pltpu.VMEM((_TOTAL,), jnp.float32),
            pltpu.VMEM((_SHUFFLE_CHANNEL,), jnp.float32),
        ],
    )
    def k(x_hbm, idx_hbm, o_hbm, idx_v, buf_v, g_v):
        wid = lax.axis_index("s") * _NC + lax.axis_index("c")
        base = wid * _RPW
        pltpu.sync_copy(idx_hbm, idx_v)

        @pl.loop(0, _RPW)
        def _row(i):
            row = base + i
            pltpu.sync_copy(x_hbm.at[row], buf_v)

            @plsc.parallel_loop(0, _SHUFFLE_CHANNEL, step=_L, unroll=8)
            def _gather(j):
                iv = idx_v[pl.ds(j, _L)]
                g_v[pl.ds(j, _L)] = plsc.load_gather(buf_v, [iv])

            @plsc.parallel_loop(0, _SHUFFLE_CHANNEL, step=_L, unroll=8)
            def _scatter(j):
                sv = idx_v[pl.ds(_SHUFFLE_CHANNEL + j, _L)]
                plsc.store_scatter(buf_v, [sv], g_v[pl.ds(j, _L)])

            pltpu.sync_copy(buf_v, o_hbm.at[row])

    return k(x2d, idx)


def kernel(input):
    x2d = input.reshape(_ROWS, _TOTAL)
    out = _sc_shuffle(x2d, jnp.asarray(_IDX))
    return out.reshape(input.shape)
